# Initial kernel scaffold; baseline (speedup 1.0000x reference)
import jax
import jax.numpy as jnp
from jax.experimental import pallas as pl


def kernel(x, top_n_indices, bottom_n_indices, W_self, att_src_self, att_dst_self, b_self, imp_w_self, imp_b_self, W_top, att_src_top, att_dst_top, b_top, imp_w_top, imp_b_top, W_bottom, att_src_bottom, att_dst_bottom, b_bottom, imp_w_bottom, imp_b_bottom, ln_gamma, ln_beta):
    raise NotImplementedError("write your pallas kernel here")


# speedup vs baseline: 13.3549x; 13.3549x over previous
import functools

import jax
import jax.numpy as jnp
from jax.experimental import pallas as pl
from jax.experimental.pallas import tpu as pltpu

_BLK = 400


def _head_mat(att):
    H, F = att.shape
    return (jnp.eye(H, dtype=att.dtype)[:, None, :] * att[:, :, None]).reshape(H * F, H)


def _proj_body(x_ref, wbig_ref, amat_ref, h_ref, a_ref):
    x = x_ref[...]
    h_ref[...] = jnp.dot(x, wbig_ref[...], preferred_element_type=jnp.float32)
    a_ref[...] = jnp.dot(x, amat_ref[...], preferred_element_type=jnp.float32)


def _epi_body(x_ref, hself_ref, ntop_ref, dtop_ref, nbot_ref, dbot_ref,
              consts_ref, out_ref, iw_ref):
    x = x_ref[...]
    c = consts_ref[...]
    H = 8
    F = 16
    rep = lambda d: jnp.repeat(d, F, axis=1)
    self_out = hself_ref[...] + c[0][None, :]
    top_out = ntop_ref[...] / rep(dtop_ref[...]) + c[1][None, :]
    bot_out = nbot_ref[...] / rep(dbot_ref[...]) + c[2][None, :]
    imp = jnp.stack([
        jnp.sum(self_out * c[3][None, :], axis=1) + c[8, 0],
        jnp.sum(top_out * c[4][None, :], axis=1) + c[8, 1],
        jnp.sum(bot_out * c[5][None, :], axis=1) + c[8, 2],
    ], axis=1)
    imp = imp - jnp.max(imp, axis=1, keepdims=True)
    eimp = jnp.exp(imp)
    iw = eimp / jnp.sum(eimp, axis=1, keepdims=True)
    out = (iw[:, 0:1] * self_out + iw[:, 1:2] * top_out + iw[:, 2:3] * bot_out)
    out = 0.5 * out * (1.0 + jax.lax.erf(out * (2.0 ** -0.5)))
    y = out + x
    mu = jnp.mean(y, axis=1, keepdims=True)
    yc = y - mu
    var = jnp.mean(yc * yc, axis=1, keepdims=True)
    out_ref[...] = yc * jax.lax.rsqrt(var + 1e-5) * c[6][None, :] + c[7][None, :]
    iw_ref[...] = iw


def _edge_accumulate(h, a_s, a_d, idx):
    N, Dm = h.shape
    K = idx.shape[1]
    e = jax.nn.leaky_relu(a_s[:, None, :] + a_d[idx], 0.2)
    w = jnp.exp(e)
    flat = idx.reshape(-1)
    den = jnp.zeros_like(a_s).at[flat].add(w.reshape(-1, a_s.shape[1]))
    msg = jnp.repeat(w, Dm // a_s.shape[1], axis=2) * h[:, None, :]
    num = jnp.zeros_like(h).at[flat].add(msg.reshape(-1, Dm))
    return num, den


def kernel(x, top_n_indices, bottom_n_indices,
           W_self, att_src_self, att_dst_self, b_self, imp_w_self, imp_b_self,
           W_top, att_src_top, att_dst_top, b_top, imp_w_top, imp_b_top,
           W_bottom, att_src_bottom, att_dst_bottom, b_bottom, imp_w_bottom, imp_b_bottom,
           ln_gamma, ln_beta):
    N, Dm = x.shape
    H, F = att_src_top.shape
    K = top_n_indices.shape[1]
    idx_top = top_n_indices.astype(jnp.int32)
    idx_bot = bottom_n_indices.astype(jnp.int32)

    wbig = jnp.concatenate([W_self, W_top, W_bottom], axis=1)
    amat = jnp.concatenate([
        W_top @ _head_mat(att_src_top), W_top @ _head_mat(att_dst_top),
        W_bottom @ _head_mat(att_src_bottom), W_bottom @ _head_mat(att_dst_bottom),
    ], axis=1)

    grid = N // _BLK
    hcat, avals = pl.pallas_call(
        _proj_body,
        grid=(grid,),
        in_specs=[
            pl.BlockSpec((_BLK, Dm), lambda i: (i, 0)),
            pl.BlockSpec((Dm, 3 * Dm), lambda i: (0, 0)),
            pl.BlockSpec((Dm, 4 * H), lambda i: (0, 0)),
        ],
        out_specs=[
            pl.BlockSpec((_BLK, 3 * Dm), lambda i: (i, 0)),
            pl.BlockSpec((_BLK, 4 * H), lambda i: (i, 0)),
        ],
        out_shape=[
            jax.ShapeDtypeStruct((N, 3 * Dm), jnp.float32),
            jax.ShapeDtypeStruct((N, 4 * H), jnp.float32),
        ],
    )(x, wbig, amat)

    h_self = hcat[:, :Dm]
    h_top = hcat[:, Dm:2 * Dm]
    h_bot = hcat[:, 2 * Dm:]
    as_top, ad_top = avals[:, :H], avals[:, H:2 * H]
    as_bot, ad_bot = avals[:, 2 * H:3 * H], avals[:, 3 * H:]

    num_top, den_top = _edge_accumulate(h_top, as_top, ad_top, idx_top)
    num_bot, den_bot = _edge_accumulate(h_bot, as_bot, ad_bot, idx_bot)

    w_self_top = jnp.exp(jax.nn.leaky_relu(as_top + ad_top, 0.2))
    w_self_bot = jnp.exp(jax.nn.leaky_relu(as_bot + ad_bot, 0.2))
    num_top = num_top + jnp.repeat(w_self_top, F, axis=1) * h_top
    den_top = den_top + w_self_top
    num_bot = num_bot + jnp.repeat(w_self_bot, F, axis=1) * h_bot
    den_bot = den_bot + w_self_bot

    consts = jnp.stack([
        b_self, b_top, b_bottom,
        imp_w_self[:, 0], imp_w_top[:, 0], imp_w_bottom[:, 0],
        ln_gamma, ln_beta,
        jnp.concatenate([imp_b_self, imp_b_top, imp_b_bottom,
                         jnp.zeros((Dm - 3,), jnp.float32)]),
    ], axis=0)

    out, iw = pl.pallas_call(
        _epi_body,
        grid=(grid,),
        in_specs=[
            pl.BlockSpec((_BLK, Dm), lambda i: (i, 0)),
            pl.BlockSpec((_BLK, Dm), lambda i: (i, 0)),
            pl.BlockSpec((_BLK, Dm), lambda i: (i, 0)),
            pl.BlockSpec((_BLK, H), lambda i: (i, 0)),
            pl.BlockSpec((_BLK, Dm), lambda i: (i, 0)),
            pl.BlockSpec((_BLK, H), lambda i: (i, 0)),
            pl.BlockSpec((9, Dm), lambda i: (0, 0)),
        ],
        out_specs=[
            pl.BlockSpec((_BLK, Dm), lambda i: (i, 0)),
            pl.BlockSpec((_BLK, 3), lambda i: (i, 0)),
        ],
        out_shape=[
            jax.ShapeDtypeStruct((N, Dm), jnp.float32),
            jax.ShapeDtypeStruct((N, 3), jnp.float32),
        ],
    )(x, h_self, num_top, den_top, num_bot, den_bot, consts)
    return (out, iw)


# speedup vs baseline: 34.0839x; 2.5522x over previous
import functools

import jax
import jax.numpy as jnp
from jax import lax
from jax.experimental import pallas as pl
from jax.experimental.pallas import tpu as pltpu
from jax.experimental.pallas import tpu_sc as plsc

_BLK = 400
_EBLK = 200
_NC = 2
_NS = 16
_C = 8


def _head_mat(att):
    H, F = att.shape
    return (jnp.eye(H, dtype=att.dtype)[:, None, :] * att[:, :, None]).reshape(H * F, H)


def _proj_body(x_ref, wcat_ref, acat_ref, hself_ref, h2_ref, as2_ref, ad2_ref):
    x = x_ref[...]
    D = x.shape[1]
    w = wcat_ref[...]
    hself_ref[...] = jnp.dot(x, w[:, :D], preferred_element_type=jnp.float32)
    h2_ref[0] = jnp.dot(x, w[:, D:2 * D], preferred_element_type=jnp.float32)
    h2_ref[1] = jnp.dot(x, w[:, 2 * D:], preferred_element_type=jnp.float32)
    a = jnp.dot(x, acat_ref[...], preferred_element_type=jnp.float32)
    H = a.shape[1] // 4
    as2_ref[0] = a[:, :H]
    ad2_ref[0] = a[:, H:2 * H]
    as2_ref[1] = a[:, 2 * H:3 * H]
    ad2_ref[1] = a[:, 3 * H:]


def _sc_body(h2r, as2r, adpk, idxr, acc_out, den_out, w_out,
             acc_sh, tab, h_chunk, as_chunk, idx2d, wbuf, msg, zidx):
    H = 8
    K = 32
    N = tab.shape[0] // 4
    HALF = N // 2
    rows_per_tile = acc_sh.shape[0] // _NS
    n_chunks = N // _C
    chunks_per_tile = -(-n_chunks // _NS)
    wb_step = rows_per_tile // 5

    c = lax.axis_index("c")
    s = lax.axis_index("s")
    zero16 = jnp.zeros((16,), jnp.float32)
    iota16 = lax.iota(jnp.int32, 16)

    pltpu.sync_copy(adpk.at[c], tab)

    def _zrow(r, carry):
        for j in range(msg.shape[1] // 16):
            msg[r, pl.ds(j * 16, 16)] = zero16
        return carry
    lax.fori_loop(0, msg.shape[0], _zrow, 0)

    def _zacc(k, carry):
        rb = s * rows_per_tile + k * 32
        zidx[pl.ds(0, 16)] = rb + iota16
        zidx[pl.ds(16, 16)] = rb + 16 + iota16
        pltpu.sync_copy(msg, acc_sh.at[zidx])
        return carry
    lax.fori_loop(0, rows_per_tile // 32, _zacc, 0)
    plsc.subcore_barrier()

    def _chunk(t, carry):
        cid = s + t * _NS

        @pl.when(cid < n_chunks)
        def _():
            row = c * n_chunks + cid
            pltpu.sync_copy(idxr.at[row], idx2d)
            pltpu.sync_copy(h2r.at[row], h_chunk)
            pltpu.sync_copy(as2r.at[row], as_chunk)

            def _node(ni, carry2):
                for g in range(2):
                    jvec = idx2d[ni, pl.ds(g * 16, 16)]
                    ebase = ni * K + g * 16
                    for hp in range(4):
                        word = plsc.load_gather(tab, [jvec * 4 + hp])
                        wi = plsc.bitcast(word, jnp.int32)
                        lo = plsc.bitcast(wi << 16, jnp.float32)
                        hi = plsc.bitcast(wi & jnp.int32(-65536), jnp.float32)
                        for hh, adv in ((2 * hp, lo), (2 * hp + 1, hi)):
                            asb = plsc.load_gather(
                                as_chunk, [jnp.full((16,), ni * H + hh,
                                                    jnp.int32)])
                            asum = asb + adv
                            e = jnp.where(asum >= 0, asum, asum * 0.2)
                            plsc.store_scatter(
                                wbuf, [(ebase + iota16) * H + hh], jnp.exp(e))
                for hh in range(H):
                    hv = h_chunk[pl.ds(ni * 128 + hh * 16, 16)]
                    for er in range(K):
                        wb = plsc.load_gather(
                            wbuf, [jnp.full((16,), (ni * K + er) * H + hh,
                                            jnp.int32)])
                        msg[er, pl.ds(hh * 16, 16)] = wb * hv
                pltpu.sync_copy(msg, acc_sh.at[idx2d.at[ni]], add=True)
                return carry2

            lax.fori_loop(0, _C, _node, 0)
            pltpu.sync_copy(wbuf, w_out.at[row])
        return carry

    lax.fori_loop(0, chunks_per_tile, _chunk, 0)
    plsc.subcore_barrier()

    def _rdb(k, carry):
        rb = s * rows_per_tile + k * 32
        zidx[pl.ds(0, 16)] = rb + iota16
        zidx[pl.ds(16, 16)] = rb + 16 + iota16
        pltpu.sync_copy(acc_sh.at[zidx], msg)
        pltpu.sync_copy(msg, acc_out.at[c * _NS * 20 + s * 20 + k])
        return carry
    lax.fori_loop(0, rows_per_tile // 32, _rdb, 0)

    for p in range(2):
        def _zden(i, carry):
            tab[pl.ds(i * 16, 16)] = zero16
            return carry
        lax.fori_loop(0, tab.shape[0] // 16, _zden, 0)

        def _chunk_den(t, carry):
            cid = s + t * _NS

            @pl.when(cid < n_chunks)
            def _():
                row = c * n_chunks + cid
                pltpu.sync_copy(idxr.at[row], idx2d)
                pltpu.sync_copy(w_out.at[row], wbuf)

                def _node(ni, carry2):
                    for g in range(2):
                        jvec = idx2d[ni, pl.ds(g * 16, 16)]
                        ebase = ni * K + g * 16
                        inh = jnp.logical_and(jvec >= p * HALF,
                                              jvec < (p + 1) * HALF)
                        jl = jnp.where(inh, jvec - p * HALF, 0)
                        cnt, last = plsc.scan_count(jvec)
                        m = jnp.logical_and(last, inh)
                        cntf = cnt.astype(jnp.float32)
                        jbase = jl * H
                        for hh in range(H):
                            w = plsc.load_gather(
                                wbuf, [(ebase + iota16) * H + hh])
                            plsc.addupdate_scatter(tab, [jbase + hh],
                                                   w * cntf, mask=m)
                    return carry2

                lax.fori_loop(0, _C, _node, 0)
            return carry

        lax.fori_loop(0, chunks_per_tile, _chunk_den, 0)
        pltpu.sync_copy(tab, den_out.at[p * 2 * _NS + c * _NS + s])


def _edge_accumulate_sc(h2r, as2r, adpk, idxr, N):
    npad = _NS * 640
    n_chunks = N // _C
    mesh = plsc.VectorSubcoreMesh(core_axis_name="c", subcore_axis_name="s")
    return pl.kernel(
        _sc_body,
        mesh=mesh,
        compiler_params=pltpu.CompilerParams(needs_layout_passes=False),
        out_type=[
            jax.ShapeDtypeStruct((_NC * npad // 32, 32, 128), jnp.float32),
            jax.ShapeDtypeStruct((4 * _NS, N * 4), jnp.float32),
            jax.ShapeDtypeStruct((_NC * n_chunks, _C * 32 * 8), jnp.float32),
        ],
        scratch_types=[
            pltpu.VMEM_SHARED((npad, 128), jnp.float32),
            pltpu.VMEM((N * 4,), jnp.float32),
            pltpu.VMEM((_C * 128,), jnp.float32),
            pltpu.VMEM((_C * 8,), jnp.float32),
            pltpu.VMEM((_C, 32), jnp.int32),
            pltpu.VMEM((_C * 32 * 8,), jnp.float32),
            pltpu.VMEM((32, 128), jnp.float32),
            pltpu.VMEM((32,), jnp.int32),
        ],
    )(h2r, as2r, adpk, idxr)


def _epi_body(x_ref, hself_ref, h2_ref, as2_ref, ad2_ref,
              acc_ref, den_ref, consts_ref, out_ref, iw_ref):
    x = x_ref[...]
    c = consts_ref[...]
    F = 16
    rep = lambda d: jnp.repeat(d, F, axis=1)
    self_out = hself_ref[...] + c[0][None, :]
    denp = den_ref[0]
    outs = []
    for b in range(2):
        w_self = jnp.exp(jax.nn.leaky_relu(as2_ref[b] + ad2_ref[b], 0.2))
        den_sc = jnp.sum(denp[b * _NS:(b + 1) * _NS], axis=0)
        num = acc_ref[b] + rep(w_self) * h2_ref[b]
        den = den_sc + w_self
        outs.append(num / rep(den) + c[1 + b][None, :])
    top_out, bot_out = outs
    imp = jnp.stack([
        jnp.sum(self_out * c[3][None, :], axis=1) + c[8, 0],
        jnp.sum(top_out * c[4][None, :], axis=1) + c[8, 1],
        jnp.sum(bot_out * c[5][None, :], axis=1) + c[8, 2],
    ], axis=1)
    imp = imp - jnp.max(imp, axis=1, keepdims=True)
    eimp = jnp.exp(imp)
    iw = eimp / jnp.sum(eimp, axis=1, keepdims=True)
    out = (iw[:, 0:1] * self_out + iw[:, 1:2] * top_out + iw[:, 2:3] * bot_out)
    out = 0.5 * out * (1.0 + jax.lax.erf(out * (2.0 ** -0.5)))
    y = out + x
    mu = jnp.mean(y, axis=1, keepdims=True)
    yc = y - mu
    var = jnp.mean(yc * yc, axis=1, keepdims=True)
    out_ref[...] = yc * jax.lax.rsqrt(var + 1e-5) * c[6][None, :] + c[7][None, :]
    iw_ref[...] = iw


def kernel(x, top_n_indices, bottom_n_indices,
           W_self, att_src_self, att_dst_self, b_self, imp_w_self, imp_b_self,
           W_top, att_src_top, att_dst_top, b_top, imp_w_top, imp_b_top,
           W_bottom, att_src_bottom, att_dst_bottom, b_bottom, imp_w_bottom, imp_b_bottom,
           ln_gamma, ln_beta):
    N, Dm = x.shape
    H, F = att_src_top.shape
    K = top_n_indices.shape[1]
    n_chunks = N // _C
    idxr = jnp.stack([top_n_indices.astype(jnp.int32),
                      bottom_n_indices.astype(jnp.int32)]).reshape(
                          _NC * n_chunks, _C, K)

    wcat = jnp.concatenate([W_self, W_top, W_bottom], axis=1)
    acat = jnp.concatenate([
        W_top @ _head_mat(att_src_top), W_top @ _head_mat(att_dst_top),
        W_bottom @ _head_mat(att_src_bottom), W_bottom @ _head_mat(att_dst_bottom),
    ], axis=1)

    grid = N // _BLK
    hself, h2, as2, ad2 = pl.pallas_call(
        _proj_body,
        grid=(grid,),
        in_specs=[
            pl.BlockSpec((_BLK, Dm), lambda i: (i, 0)),
            pl.BlockSpec((Dm, 3 * Dm), lambda i: (0, 0)),
            pl.BlockSpec((Dm, 4 * H), lambda i: (0, 0)),
        ],
        out_specs=[
            pl.BlockSpec((_BLK, Dm), lambda i: (i, 0)),
            pl.BlockSpec((_NC, _BLK, Dm), lambda i: (0, i, 0)),
            pl.BlockSpec((_NC, _BLK, H), lambda i: (0, i, 0)),
            pl.BlockSpec((_NC, _BLK, H), lambda i: (0, i, 0)),
        ],
        out_shape=[
            jax.ShapeDtypeStruct((N, Dm), jnp.float32),
            jax.ShapeDtypeStruct((_NC, N, Dm), jnp.float32),
            jax.ShapeDtypeStruct((_NC, N, H), jnp.float32),
            jax.ShapeDtypeStruct((_NC, N, H), jnp.float32),
        ],
    )(x, wcat, acat)

    ad_u16 = jax.lax.bitcast_convert_type(ad2.astype(jnp.bfloat16), jnp.uint16)
    adpk = jax.lax.bitcast_convert_type(
        (ad_u16[:, :, 0::2].astype(jnp.uint32)
         | (ad_u16[:, :, 1::2].astype(jnp.uint32) << 16)),
        jnp.float32).reshape(_NC, N * 4)

    acc, den_part, _w_scratch = _edge_accumulate_sc(
        h2.reshape(_NC * n_chunks, _C * Dm), as2.reshape(_NC * n_chunks, _C * H),
        adpk, idxr, N)

    consts = jnp.stack([
        b_self, b_top, b_bottom,
        imp_w_self[:, 0], imp_w_top[:, 0], imp_w_bottom[:, 0],
        ln_gamma, ln_beta,
        jnp.concatenate([imp_b_self, imp_b_top, imp_b_bottom,
                         jnp.zeros((Dm - 3,), jnp.float32)]),
    ], axis=0)

    egrid = N // _EBLK
    hb = (N // 2) // _EBLK
    out, iw = pl.pallas_call(
        _epi_body,
        grid=(egrid,),
        in_specs=[
            pl.BlockSpec((_EBLK, Dm), lambda i: (i, 0)),
            pl.BlockSpec((_EBLK, Dm), lambda i: (i, 0)),
            pl.BlockSpec((_NC, _EBLK, Dm), lambda i: (0, i, 0)),
            pl.BlockSpec((_NC, _EBLK, H), lambda i: (0, i, 0)),
            pl.BlockSpec((_NC, _EBLK, H), lambda i: (0, i, 0)),
            pl.BlockSpec((_NC, _EBLK, Dm), lambda i: (0, i, 0)),
            pl.BlockSpec((1, 2 * _NS, _EBLK, H), lambda i: (i // hb, 0, i % hb, 0)),
            pl.BlockSpec((9, Dm), lambda i: (0, 0)),
        ],
        out_specs=[
            pl.BlockSpec((_EBLK, Dm), lambda i: (i, 0)),
            pl.BlockSpec((_EBLK, 3), lambda i: (i, 0)),
        ],
        out_shape=[
            jax.ShapeDtypeStruct((N, Dm), jnp.float32),
            jax.ShapeDtypeStruct((N, 3), jnp.float32),
        ],
    )(x, hself, h2, as2, ad2, acc.reshape(_NC, _NS * 640, Dm)[:, :N, :],
      den_part.reshape(2, 2 * _NS, N // 2, H), consts)
    return (out, iw)


# speedup vs baseline: 35.7868x; 1.0500x over previous
import functools

import jax
import jax.numpy as jnp
from jax import lax
from jax.experimental import pallas as pl
from jax.experimental.pallas import tpu as pltpu
from jax.experimental.pallas import tpu_sc as plsc

_BLK = 400
_EBLK = 200
_NC = 2
_NS = 16
_C = 8


def _head_mat(att):
    H, F = att.shape
    return (jnp.eye(H, dtype=att.dtype)[:, None, :] * att[:, :, None]).reshape(H * F, H)


def _proj_body(x_ref, wcat_ref, acat_ref, hself_ref, h2_ref, as2_ref, ad2_ref):
    x = x_ref[...]
    D = x.shape[1]
    w = wcat_ref[...]
    hself_ref[...] = jnp.dot(x, w[:, :D], preferred_element_type=jnp.float32)
    h2_ref[0] = jnp.dot(x, w[:, D:2 * D], preferred_element_type=jnp.float32)
    h2_ref[1] = jnp.dot(x, w[:, 2 * D:], preferred_element_type=jnp.float32)
    a = jnp.dot(x, acat_ref[...], preferred_element_type=jnp.float32)
    H = a.shape[1] // 4
    as2_ref[0] = a[:, :H]
    ad2_ref[0] = a[:, H:2 * H]
    as2_ref[1] = a[:, 2 * H:3 * H]
    ad2_ref[1] = a[:, 3 * H:]


def _sc_body(h2r, as2r, adpk, idxr, acc_out, den_out, w_out,
             acc_sh, tab, h_chunk, as_chunk, idx2d, wbuf, msg, zidx, isem):
    H = 8
    K = 32
    N = tab.shape[0] // 4
    HALF = N // 2
    rows_per_tile = acc_sh.shape[0] // _NS
    n_chunks = N // _C
    chunks_per_tile = -(-n_chunks // _NS)
    wb_step = rows_per_tile // 5

    c = lax.axis_index("c")
    s = lax.axis_index("s")
    zero16 = jnp.zeros((16,), jnp.float32)
    iota16 = lax.iota(jnp.int32, 16)

    pltpu.sync_copy(adpk.at[c], tab)

    def _zrow(r, carry):
        for j in range(msg.shape[1] // 16):
            msg[r, pl.ds(j * 16, 16)] = zero16
        return carry
    lax.fori_loop(0, msg.shape[0], _zrow, 0)

    def _zacc(k, carry):
        rb = s * rows_per_tile + k * 32
        zidx[pl.ds(0, 16)] = rb + iota16
        zidx[pl.ds(16, 16)] = rb + 16 + iota16
        pltpu.sync_copy(msg, acc_sh.at[zidx])
        return carry
    lax.fori_loop(0, rows_per_tile // 32, _zacc, 0)
    plsc.subcore_barrier()

    def _chunk(t, carry):
        cid = s + t * _NS

        @pl.when(cid < n_chunks)
        def _():
            row = c * n_chunks + cid
            pltpu.async_copy(idxr.at[row], idx2d, isem)
            pltpu.async_copy(h2r.at[row], h_chunk, isem)
            pltpu.async_copy(as2r.at[row], as_chunk, isem)
            pltpu.make_async_copy(idxr.at[row], idx2d, isem).wait()
            pltpu.make_async_copy(h2r.at[row], h_chunk, isem).wait()
            pltpu.make_async_copy(as2r.at[row], as_chunk, isem).wait()

            def _node(ni, carry2):
                for g in range(2):
                    jvec = idx2d[ni, pl.ds(g * 16, 16)]
                    ebase = ni * K + g * 16
                    for hp in range(4):
                        word = plsc.load_gather(tab, [jvec * 4 + hp])
                        wi = plsc.bitcast(word, jnp.int32)
                        lo = plsc.bitcast(wi << 16, jnp.float32)
                        hi = plsc.bitcast(wi & jnp.int32(-65536), jnp.float32)
                        for hh, adv in ((2 * hp, lo), (2 * hp + 1, hi)):
                            asb = plsc.load_gather(
                                as_chunk, [jnp.full((16,), ni * H + hh,
                                                    jnp.int32)])
                            asum = asb + adv
                            e = jnp.where(asum >= 0, asum, asum * 0.2)
                            plsc.store_scatter(
                                wbuf, [(ebase + iota16) * H + hh], jnp.exp(e))
                for hh in range(H):
                    hv = h_chunk[pl.ds(ni * 128 + hh * 16, 16)]
                    for er in range(K):
                        wb = plsc.load_gather(
                            wbuf, [jnp.full((16,), (ni * K + er) * H + hh,
                                            jnp.int32)])
                        msg[er, pl.ds(hh * 16, 16)] = wb * hv
                pltpu.sync_copy(msg, acc_sh.at[idx2d.at[ni]], add=True)
                return carry2

            lax.fori_loop(0, _C, _node, 0)
            pltpu.sync_copy(wbuf, w_out.at[row])
        return carry

    lax.fori_loop(0, chunks_per_tile, _chunk, 0)
    plsc.subcore_barrier()

    def _rdb(k, carry):
        rb = s * rows_per_tile + k * 32
        zidx[pl.ds(0, 16)] = rb + iota16
        zidx[pl.ds(16, 16)] = rb + 16 + iota16
        pltpu.sync_copy(acc_sh.at[zidx], msg)
        pltpu.sync_copy(msg, acc_out.at[c * _NS * 20 + s * 20 + k])
        return carry
    lax.fori_loop(0, rows_per_tile // 32, _rdb, 0)

    for p in range(2):
        def _zden(i, carry):
            tab[pl.ds(i * 16, 16)] = zero16
            return carry
        lax.fori_loop(0, tab.shape[0] // 16, _zden, 0)

        def _chunk_den(t, carry):
            cid = s + t * _NS

            @pl.when(cid < n_chunks)
            def _():
                row = c * n_chunks + cid
                pltpu.async_copy(idxr.at[row], idx2d, isem)
                pltpu.async_copy(w_out.at[row], wbuf, isem)
                pltpu.make_async_copy(idxr.at[row], idx2d, isem).wait()
                pltpu.make_async_copy(w_out.at[row], wbuf, isem).wait()

                def _node(ni, carry2):
                    for g in range(2):
                        jvec = idx2d[ni, pl.ds(g * 16, 16)]
                        ebase = ni * K + g * 16
                        inh = jnp.logical_and(jvec >= p * HALF,
                                              jvec < (p + 1) * HALF)
                        jl = jnp.where(inh, jvec - p * HALF, 0)
                        cnt, last = plsc.scan_count(jvec)
                        m = jnp.logical_and(last, inh)
                        cntf = cnt.astype(jnp.float32)
                        jbase = jl * H
                        for hh in range(H):
                            w = plsc.load_gather(
                                wbuf, [(ebase + iota16) * H + hh])
                            plsc.addupdate_scatter(tab, [jbase + hh],
                                                   w * cntf, mask=m)
                    return carry2

                lax.fori_loop(0, _C, _node, 0)
            return carry

        lax.fori_loop(0, chunks_per_tile, _chunk_den, 0)
        pltpu.sync_copy(tab, den_out.at[p * 2 * _NS + c * _NS + s])


def _edge_accumulate_sc(h2r, as2r, adpk, idxr, N):
    npad = _NS * 640
    n_chunks = N // _C
    mesh = plsc.VectorSubcoreMesh(core_axis_name="c", subcore_axis_name="s")
    return pl.kernel(
        _sc_body,
        mesh=mesh,
        compiler_params=pltpu.CompilerParams(needs_layout_passes=False),
        out_type=[
            jax.ShapeDtypeStruct((_NC * npad // 32, 32, 128), jnp.float32),
            jax.ShapeDtypeStruct((4 * _NS, N * 4), jnp.float32),
            jax.ShapeDtypeStruct((_NC * n_chunks, _C * 32 * 8), jnp.float32),
        ],
        scratch_types=[
            pltpu.VMEM_SHARED((npad, 128), jnp.float32),
            pltpu.VMEM((N * 4,), jnp.float32),
            pltpu.VMEM((_C * 128,), jnp.float32),
            pltpu.VMEM((_C * 8,), jnp.float32),
            pltpu.VMEM((_C, 32), jnp.int32),
            pltpu.VMEM((_C * 32 * 8,), jnp.float32),
            pltpu.VMEM((32, 128), jnp.float32),
            pltpu.VMEM((32,), jnp.int32),
            pltpu.SemaphoreType.DMA,
        ],
    )(h2r, as2r, adpk, idxr)


def _epi_body(x_ref, hself_ref, h2_ref, as2_ref, ad2_ref,
              acc_ref, den_ref, consts_ref, out_ref, iw_ref):
    x = x_ref[...]
    c = consts_ref[...]
    F = 16
    rep = lambda d: jnp.repeat(d, F, axis=1)
    self_out = hself_ref[...] + c[0][None, :]
    denp = den_ref[0]
    outs = []
    for b in range(2):
        w_self = jnp.exp(jax.nn.leaky_relu(as2_ref[b] + ad2_ref[b], 0.2))
        den_sc = jnp.sum(denp[b * _NS:(b + 1) * _NS], axis=0)
        num = acc_ref[b] + rep(w_self) * h2_ref[b]
        den = den_sc + w_self
        outs.append(num / rep(den) + c[1 + b][None, :])
    top_out, bot_out = outs
    imp = jnp.stack([
        jnp.sum(self_out * c[3][None, :], axis=1) + c[8, 0],
        jnp.sum(top_out * c[4][None, :], axis=1) + c[8, 1],
        jnp.sum(bot_out * c[5][None, :], axis=1) + c[8, 2],
    ], axis=1)
    imp = imp - jnp.max(imp, axis=1, keepdims=True)
    eimp = jnp.exp(imp)
    iw = eimp / jnp.sum(eimp, axis=1, keepdims=True)
    out = (iw[:, 0:1] * self_out + iw[:, 1:2] * top_out + iw[:, 2:3] * bot_out)
    out = 0.5 * out * (1.0 + jax.lax.erf(out * (2.0 ** -0.5)))
    y = out + x
    mu = jnp.mean(y, axis=1, keepdims=True)
    yc = y - mu
    var = jnp.mean(yc * yc, axis=1, keepdims=True)
    out_ref[...] = yc * jax.lax.rsqrt(var + 1e-5) * c[6][None, :] + c[7][None, :]
    iw_ref[...] = iw


def kernel(x, top_n_indices, bottom_n_indices,
           W_self, att_src_self, att_dst_self, b_self, imp_w_self, imp_b_self,
           W_top, att_src_top, att_dst_top, b_top, imp_w_top, imp_b_top,
           W_bottom, att_src_bottom, att_dst_bottom, b_bottom, imp_w_bottom, imp_b_bottom,
           ln_gamma, ln_beta):
    N, Dm = x.shape
    H, F = att_src_top.shape
    K = top_n_indices.shape[1]
    n_chunks = N // _C
    idxr = jnp.stack([top_n_indices.astype(jnp.int32),
                      bottom_n_indices.astype(jnp.int32)]).reshape(
                          _NC * n_chunks, _C, K)

    wcat = jnp.concatenate([W_self, W_top, W_bottom], axis=1)
    acat = jnp.concatenate([
        W_top @ _head_mat(att_src_top), W_top @ _head_mat(att_dst_top),
        W_bottom @ _head_mat(att_src_bottom), W_bottom @ _head_mat(att_dst_bottom),
    ], axis=1)

    grid = N // _BLK
    hself, h2, as2, ad2 = pl.pallas_call(
        _proj_body,
        grid=(grid,),
        in_specs=[
            pl.BlockSpec((_BLK, Dm), lambda i: (i, 0)),
            pl.BlockSpec((Dm, 3 * Dm), lambda i: (0, 0)),
            pl.BlockSpec((Dm, 4 * H), lambda i: (0, 0)),
        ],
        out_specs=[
            pl.BlockSpec((_BLK, Dm), lambda i: (i, 0)),
            pl.BlockSpec((_NC, _BLK, Dm), lambda i: (0, i, 0)),
            pl.BlockSpec((_NC, _BLK, H), lambda i: (0, i, 0)),
            pl.BlockSpec((_NC, _BLK, H), lambda i: (0, i, 0)),
        ],
        out_shape=[
            jax.ShapeDtypeStruct((N, Dm), jnp.float32),
            jax.ShapeDtypeStruct((_NC, N, Dm), jnp.float32),
            jax.ShapeDtypeStruct((_NC, N, H), jnp.float32),
            jax.ShapeDtypeStruct((_NC, N, H), jnp.float32),
        ],
    )(x, wcat, acat)

    ad_u16 = jax.lax.bitcast_convert_type(ad2.astype(jnp.bfloat16), jnp.uint16)
    adpk = jax.lax.bitcast_convert_type(
        (ad_u16[:, :, 0::2].astype(jnp.uint32)
         | (ad_u16[:, :, 1::2].astype(jnp.uint32) << 16)),
        jnp.float32).reshape(_NC, N * 4)

    acc, den_part, _w_scratch = _edge_accumulate_sc(
        h2.reshape(_NC * n_chunks, _C * Dm), as2.reshape(_NC * n_chunks, _C * H),
        adpk, idxr, N)

    consts = jnp.stack([
        b_self, b_top, b_bottom,
        imp_w_self[:, 0], imp_w_top[:, 0], imp_w_bottom[:, 0],
        ln_gamma, ln_beta,
        jnp.concatenate([imp_b_self, imp_b_top, imp_b_bottom,
                         jnp.zeros((Dm - 3,), jnp.float32)]),
    ], axis=0)

    egrid = N // _EBLK
    hb = (N // 2) // _EBLK
    out, iw = pl.pallas_call(
        _epi_body,
        grid=(egrid,),
        in_specs=[
            pl.BlockSpec((_EBLK, Dm), lambda i: (i, 0)),
            pl.BlockSpec((_EBLK, Dm), lambda i: (i, 0)),
            pl.BlockSpec((_NC, _EBLK, Dm), lambda i: (0, i, 0)),
            pl.BlockSpec((_NC, _EBLK, H), lambda i: (0, i, 0)),
            pl.BlockSpec((_NC, _EBLK, H), lambda i: (0, i, 0)),
            pl.BlockSpec((_NC, _EBLK, Dm), lambda i: (0, i, 0)),
            pl.BlockSpec((1, 2 * _NS, _EBLK, H), lambda i: (i // hb, 0, i % hb, 0)),
            pl.BlockSpec((9, Dm), lambda i: (0, 0)),
        ],
        out_specs=[
            pl.BlockSpec((_EBLK, Dm), lambda i: (i, 0)),
            pl.BlockSpec((_EBLK, 3), lambda i: (i, 0)),
        ],
        out_shape=[
            jax.ShapeDtypeStruct((N, Dm), jnp.float32),
            jax.ShapeDtypeStruct((N, 3), jnp.float32),
        ],
    )(x, hself, h2, as2, ad2, acc.reshape(_NC, _NS * 640, Dm)[:, :N, :],
      den_part.reshape(2, 2 * _NS, N // 2, H), consts)
    return (out, iw)
